# Initial kernel scaffold; baseline (speedup 1.0000x reference)
import jax
import jax.numpy as jnp
from jax.experimental import pallas as pl


def kernel(x, W, b):
    raise NotImplementedError("write your pallas kernel here")


# speedup vs baseline: 1.3564x; 1.3564x over previous
import functools

import jax
import jax.numpy as jnp
from jax.experimental import pallas as pl
from jax.experimental.pallas import tpu as pltpu

N_EXPERTS = 64
N_GROUPS = 8
GROUP_SIZE = 8
TOPK_GROUPS = 4
TOPK = 8
ROUTE_SCALE = 2.5

BM = 256


def _gate_body(x_ref, wt_ref, b_ref, w_out_ref, idx_out_ref):
    x_blk = x_ref[...]
    wt = wt_ref[...]
    acc = jnp.dot(x_blk, wt, preferred_element_type=jnp.float32)
    s = jax.nn.sigmoid(acc + b_ref[...])

    n = s.shape[0]
    neg_inf = jnp.float32(-jnp.inf)

    gs_list = []
    for g in range(N_GROUPS):
        sg = s[:, g * GROUP_SIZE:(g + 1) * GROUP_SIZE]
        m1 = jnp.max(sg, axis=1, keepdims=True)
        iota = jax.lax.broadcasted_iota(jnp.int32, (n, GROUP_SIZE), 1)
        first = jnp.min(jnp.where(sg == m1, iota, GROUP_SIZE), axis=1,
                        keepdims=True)
        m2 = jnp.max(jnp.where(iota == first, neg_inf, sg), axis=1,
                     keepdims=True)
        gs_list.append(m1 + m2)

    keep_cols = []
    for g in range(N_GROUPS):
        r = jnp.zeros((n, 1), dtype=jnp.int32)
        for h in range(N_GROUPS):
            if h == g:
                continue
            if h < g:
                c = gs_list[h] >= gs_list[g]
            else:
                c = gs_list[h] > gs_list[g]
            r = r + c.astype(jnp.int32)
        keep_cols.append(r < TOPK_GROUPS)

    sm_parts = []
    for g in range(N_GROUPS):
        sg = s[:, g * GROUP_SIZE:(g + 1) * GROUP_SIZE]
        sm_parts.append(jnp.where(keep_cols[g], sg, neg_inf))
    sm = jnp.concatenate(sm_parts, axis=1)

    iota64 = jax.lax.broadcasted_iota(jnp.int32, (n, N_EXPERTS), 1)
    idxs, vals = [], []
    for _ in range(TOPK):
        m = jnp.max(sm, axis=1, keepdims=True)
        idx = jnp.min(jnp.where(sm == m, iota64, N_EXPERTS), axis=1,
                      keepdims=True)
        idxs.append(idx)
        vals.append(m)
        sm = jnp.where(iota64 == idx, neg_inf, sm)
    indices = jnp.concatenate(idxs, axis=1)
    v = jnp.concatenate(vals, axis=1)
    w = v / jnp.sum(v, axis=1, keepdims=True) * ROUTE_SCALE

    w_out_ref[...] = w
    idx_out_ref[...] = indices


@jax.jit
def kernel(x, W, b):
    B, D = x.shape
    wt = W.T
    b2 = b.reshape(1, N_EXPERTS)
    grid = (B // BM,)
    w_out, idx_out = pl.pallas_call(
        _gate_body,
        grid=grid,
        in_specs=[
            pl.BlockSpec((BM, D), lambda i: (i, 0)),
            pl.BlockSpec((D, N_EXPERTS), lambda i: (0, 0)),
            pl.BlockSpec((1, N_EXPERTS), lambda i: (0, 0)),
        ],
        out_specs=[
            pl.BlockSpec((BM, TOPK), lambda i: (i, 0)),
            pl.BlockSpec((BM, TOPK), lambda i: (i, 0)),
        ],
        out_shape=[
            jax.ShapeDtypeStruct((B, TOPK), jnp.float32),
            jax.ShapeDtypeStruct((B, TOPK), jnp.int32),
        ],
    )(x, wt, b2)
    return w_out, idx_out


# speedup vs baseline: 4.8298x; 3.5607x over previous
import functools

import jax
import jax.numpy as jnp
from jax import lax
from jax.experimental import pallas as pl
from jax.experimental.pallas import tpu as pltpu
from jax.experimental.pallas import tpu_sc as plsc

N_EXPERTS = 64
N_GROUPS = 8
GROUP_SIZE = 8
TOPK_GROUPS = 4
TOPK = 8
ROUTE_SCALE = 2.5

NC, NS, L = 2, 16, 16
NW = NC * NS
NEG_INF = float("-inf")


def _scores_body(x_ref, wt_ref, b_ref, st_ref):
    acc = jnp.dot(x_ref[...], wt_ref[...], preferred_element_type=jnp.float32)
    s = jax.nn.sigmoid(acc + b_ref[...])
    st_ref[0] = s.T


def _tc_scores(x, wt, b2, chunk):
    B = x.shape[0]
    D = x.shape[1]
    nblk = B // chunk
    return pl.pallas_call(
        _scores_body,
        grid=(nblk,),
        in_specs=[
            pl.BlockSpec((chunk, D), lambda i: (i, 0)),
            pl.BlockSpec((D, N_EXPERTS), lambda i: (0, 0)),
            pl.BlockSpec((1, N_EXPERTS), lambda i: (0, 0)),
        ],
        out_specs=pl.BlockSpec((1, N_EXPERTS, chunk), lambda i: (i, 0, 0)),
        out_shape=jax.ShapeDtypeStruct((nblk, N_EXPERTS, chunk), jnp.float32),
    )(x, wt, b2)


def _sc_routing_body(nblk, chunk, st_hbm, w_hbm, i_hbm,
                     chunk_v, wbuf, ibuf, sem):
    wid = lax.axis_index("s") * NC + lax.axis_index("c")
    base = wid * chunk

    pltpu.async_copy(st_hbm.at[wid], chunk_v, sem).wait()

    lane = jnp.arange(L, dtype=jnp.int32)

    def block(tb, _):
        col = tb * L

        gs = []
        for g in range(N_GROUPS):
            v0 = chunk_v[g * GROUP_SIZE + 0, pl.ds(col, L)]
            v1 = chunk_v[g * GROUP_SIZE + 1, pl.ds(col, L)]
            m1 = jnp.maximum(v0, v1)
            m2 = jnp.minimum(v0, v1)
            for j in range(2, GROUP_SIZE):
                v = chunk_v[g * GROUP_SIZE + j, pl.ds(col, L)]
                m2 = jnp.maximum(m2, jnp.minimum(m1, v))
                m1 = jnp.maximum(m1, v)
            gs.append(m1 + m2)

        madd = []
        zero = jnp.zeros((L,), jnp.int32)
        one = jnp.ones((L,), jnp.int32)
        for g in range(N_GROUPS):
            r = zero
            for h in range(N_GROUPS):
                if h == g:
                    continue
                c = (gs[h] >= gs[g]) if h < g else (gs[h] > gs[g])
                r = r + jnp.where(c, one, zero)
            keep = r < TOPK_GROUPS
            madd.append(jnp.where(keep, jnp.float32(0.0), jnp.float32(NEG_INF)))

        sv = [jnp.full((L,), NEG_INF, jnp.float32)] * TOPK
        si = [zero] * TOPK
        for e in range(N_EXPERTS):
            v = chunk_v[e, pl.ds(col, L)] + madd[e // GROUP_SIZE]
            ei = jnp.full((L,), e, jnp.int32)
            c = [v > sv[j] for j in range(TOPK)]
            nsv, nsi = [], []
            for j in range(TOPK):
                if j == 0:
                    nsv.append(jnp.where(c[0], v, sv[0]))
                    nsi.append(jnp.where(c[0], ei, si[0]))
                else:
                    nsv.append(jnp.where(c[j], jnp.where(c[j - 1], sv[j - 1], v), sv[j]))
                    nsi.append(jnp.where(c[j], jnp.where(c[j - 1], si[j - 1], ei), si[j]))
            sv, si = nsv, nsi

        tot = sv[0]
        for j in range(1, TOPK):
            tot = tot + sv[j]

        for k in range(TOPK):
            wk = (sv[k] / tot) * jnp.float32(ROUTE_SCALE)
            wbuf[k, pl.ds(col, L)] = wk
            ibuf[k, pl.ds(col, L)] = si[k]
        return ()

    lax.fori_loop(0, chunk // L, block, (), unroll=1)

    pltpu.sync_copy(wbuf, w_hbm.at[:, pl.ds(base, chunk)])
    pltpu.sync_copy(ibuf, i_hbm.at[:, pl.ds(base, chunk)])


def _sc_routing(st, B, chunk):
    nblk = st.shape[0]
    mesh = plsc.VectorSubcoreMesh(core_axis_name="c", subcore_axis_name="s")
    body = functools.partial(_sc_routing_body, nblk, chunk)
    return pl.kernel(
        body,
        out_type=[
            jax.ShapeDtypeStruct((TOPK, B), jnp.float32),
            jax.ShapeDtypeStruct((TOPK, B), jnp.int32),
        ],
        mesh=mesh,
        scratch_types=[
            pltpu.VMEM((N_EXPERTS, chunk), jnp.float32),
            pltpu.VMEM((TOPK, chunk), jnp.float32),
            pltpu.VMEM((TOPK, chunk), jnp.int32),
            pltpu.SemaphoreType.DMA,
        ],
    )(st)


@jax.jit
def kernel(x, W, b):
    B = x.shape[0]
    chunk = B // NW
    wt = W.T
    b2 = b.reshape(1, N_EXPERTS)
    st = _tc_scores(x, wt, b2, chunk)
    w_t, i_t = _sc_routing(st, B, chunk)
    return w_t.T, i_t.T
